# Initial kernel scaffold; baseline (speedup 1.0000x reference)
import jax
import jax.numpy as jnp
from jax.experimental import pallas as pl


def kernel(x, edge_index, W1, b1, W2, b2, W3, b3):
    raise NotImplementedError("write your pallas kernel here")


# speedup vs baseline: 5.7417x; 5.7417x over previous
import functools

import jax
import jax.numpy as jnp
from jax import lax
from jax.experimental import pallas as pl
from jax.experimental.pallas import tpu as pltpu
from jax.experimental.pallas import tpu_sc as plsc

NC = 2
NS = 16
NW = NC * NS
CHUNK = 128
BN = 512


def _sc_mesh():
    return plsc.VectorSubcoreMesh(
        core_axis_name="c", subcore_axis_name="s", num_cores=NC,
        num_subcores=NS)


def _sc_spmm(table, srcs, dsts, zeros_nd, npad, d, k):
    zrows = npad // NS

    @functools.partial(
        pl.kernel,
        out_type=jax.ShapeDtypeStruct((NC, npad, d), jnp.float32),
        mesh=_sc_mesh(),
        scratch_types=[
            pltpu.VMEM((k, CHUNK), jnp.int32),
            pltpu.VMEM((k, CHUNK), jnp.int32),
            pltpu.VMEM((CHUNK, d), jnp.float32),
            pltpu.VMEM_SHARED((npad, d), jnp.float32),
        ],
    )
    def run(table_h, srcs_h, dsts_h, zeros_h, out_h, idx_s, idx_d, rows, acc):
        c = lax.axis_index("c")
        s = lax.axis_index("s")
        wid = c * NS + s
        pltpu.sync_copy(srcs_h.at[wid], idx_s)
        pltpu.sync_copy(dsts_h.at[wid], idx_d)
        pltpu.sync_copy(zeros_h.at[pl.ds(s * zrows, zrows)],
                        acc.at[pl.ds(s * zrows, zrows)])
        plsc.subcore_barrier()

        @pl.loop(0, k)
        def _(j):
            pltpu.sync_copy(table_h.at[idx_s.at[j]], rows)
            pltpu.sync_copy(rows, acc.at[idx_d.at[j]], add=True)

        plsc.subcore_barrier()
        pltpu.sync_copy(acc.at[pl.ds(s * zrows, zrows)],
                        out_h.at[c, pl.ds(s * zrows, zrows)])

    return run(table, srcs, dsts, zeros_nd)


def _dis_block(d0, d1, row0, n):
    deg = d0[:, :1] + d1[:, :1] + 1.0
    rows = row0 + lax.broadcasted_iota(jnp.int32, (deg.shape[0], 1), 0)
    return jnp.where(rows < n, lax.rsqrt(deg), 0.0)


def _prep_body(n, x_ref, w_ref, d0_ref, d1_ref, o_ref):
    i = pl.program_id(0)
    dis = _dis_block(d0_ref[...], d1_ref[...], i * BN, n)
    hw = jnp.dot(x_ref[...], w_ref[...], preferred_element_type=jnp.float32)
    o_ref[...] = hw * dis


def _layer_body(n, s0_ref, s1_ref, t_ref, d0_ref, d1_ref, b_ref, w_ref, o_ref):
    i = pl.program_id(0)
    dis = _dis_block(d0_ref[...], d1_ref[...], i * BN, n)
    h = dis * (s0_ref[...] + s1_ref[...] + t_ref[...]) + b_ref[...]
    h = jnp.maximum(h, 0.0)
    o_ref[...] = jnp.dot(h, w_ref[...], preferred_element_type=jnp.float32) * dis


def _final_body(n, s0_ref, s1_ref, t_ref, d0_ref, d1_ref, b_ref, o_ref):
    i = pl.program_id(0)
    dis = _dis_block(d0_ref[...], d1_ref[...], i * BN, n)
    o_ref[...] = dis * (s0_ref[...] + s1_ref[...] + t_ref[...]) + b_ref[...]


def _row_spec(d):
    return pl.BlockSpec((BN, d), lambda i: (i, 0))


def _const_spec(shape):
    return pl.BlockSpec(shape, lambda i: (0, 0))


def _tc_prep(xp, w, d0, d1, n, npad, d):
    return pl.pallas_call(
        functools.partial(_prep_body, n),
        grid=(npad // BN,),
        in_specs=[_row_spec(d), _const_spec((d, d)), _row_spec(16),
                  _row_spec(16)],
        out_specs=_row_spec(d),
        out_shape=jax.ShapeDtypeStruct((npad, d), jnp.float32),
    )(xp, w, d0, d1)


def _tc_layer(s0, s1, t, d0, d1, b, w, n, npad, d):
    return pl.pallas_call(
        functools.partial(_layer_body, n),
        grid=(npad // BN,),
        in_specs=[_row_spec(d), _row_spec(d), _row_spec(d), _row_spec(16),
                  _row_spec(16), _const_spec((1, d)), _const_spec((d, d))],
        out_specs=_row_spec(d),
        out_shape=jax.ShapeDtypeStruct((npad, d), jnp.float32),
    )(s0, s1, t, d0, d1, b, w)


def _tc_final(s0, s1, t, d0, d1, b, n, npad, d):
    return pl.pallas_call(
        functools.partial(_final_body, n),
        grid=(npad // BN,),
        in_specs=[_row_spec(d), _row_spec(d), _row_spec(d), _row_spec(16),
                  _row_spec(16), _const_spec((1, d))],
        out_specs=_row_spec(d),
        out_shape=jax.ShapeDtypeStruct((npad, d), jnp.float32),
    )(s0, s1, t, d0, d1, b)


def kernel(x, edge_index, W1, b1, W2, b2, W3, b3):
    n, d = x.shape
    e = edge_index.shape[1]
    npad = -(-n // BN) * BN
    k = -(-e // (NW * CHUNK))
    k += k % 2
    epad = NW * CHUNK * k

    pad = jnp.full((epad - e,), n, jnp.int32)
    src = jnp.concatenate([edge_index[0].astype(jnp.int32), pad])
    dst = jnp.concatenate([edge_index[1].astype(jnp.int32), pad])
    src = src.reshape(NW, k, CHUNK)
    dst = dst.reshape(NW, k, CHUNK)

    xp = jnp.zeros((npad, d), jnp.float32).at[:n].set(x)
    zeros_nd = jnp.zeros((npad, d), jnp.float32)
    ones_nd = jnp.zeros((npad, d), jnp.float32).at[:n].set(1.0)

    degp = _sc_spmm(ones_nd, src, dst, zeros_nd, npad, d, k)
    d0, d1 = degp[0, :, :16], degp[1, :, :16]
    b1r, b2r, b3r = (v.reshape(1, d) for v in (b1, b2, b3))

    t1 = _tc_prep(xp, W1, d0, d1, n, npad, d)
    s = _sc_spmm(t1, src, dst, zeros_nd, npad, d, k)
    t2 = _tc_layer(s[0], s[1], t1, d0, d1, b1r, W2, n, npad, d)
    s = _sc_spmm(t2, src, dst, zeros_nd, npad, d, k)
    t3 = _tc_layer(s[0], s[1], t2, d0, d1, b2r, W3, n, npad, d)
    s = _sc_spmm(t3, src, dst, zeros_nd, npad, d, k)
    out = _tc_final(s[0], s[1], t3, d0, d1, b3r, n, npad, d)
    return out[:n]
